# baseline (speedup 1.0000x reference)
import functools

import jax
import jax.numpy as jnp
from jax import lax
from jax.experimental import pallas as pl
from jax.experimental.pallas import tpu as pltpu
from jax.experimental.pallas import tpu_sc as plsc

N_NODES = 10000
N_EDGES = 160000
FEAT = 256
HALF = 128
OUT_F = 256

NPAD = 10112
CPT = NPAD // 16
EB = 64
RPT = 160
CH = 16
NCH = RPT // CH
EPAD = 16 * RPT * EB


_sc_mesh = plsc.VectorSubcoreMesh(core_axis_name="c", subcore_axis_name="s")


@functools.partial(
    pl.kernel,
    out_type=[
        jax.ShapeDtypeStruct((2, NPAD, HALF), jnp.float32),
        jax.ShapeDtypeStruct((NPAD,), jnp.float32),
    ],
    mesh=_sc_mesh,
    scratch_types=[
        pltpu.VMEM((CH, EB), jnp.int32),
        pltpu.VMEM((CH, EB), jnp.int32),
        pltpu.VMEM((CH, EB), jnp.int32),
        pltpu.VMEM((CH, EB), jnp.int32),
        pltpu.VMEM((EB, HALF), jnp.float32),
        pltpu.VMEM((EB, HALF), jnp.float32),
        pltpu.VMEM((EB, HALF), jnp.float32),
        pltpu.VMEM((EB,), jnp.float32),
        pltpu.VMEM((640,), jnp.float32),
        pltpu.VMEM_SHARED((NPAD, HALF), jnp.float32),
        pltpu.VMEM_SHARED((NPAD,), jnp.float32),
        pltpu.SemaphoreType.DMA,
        pltpu.SemaphoreType.DMA,
        pltpu.SemaphoreType.DMA,
        pltpu.SemaphoreType.DMA,
        pltpu.SemaphoreType.DMA,
        pltpu.SemaphoreType.DMA,
        pltpu.SemaphoreType.DMA,
        pltpu.SemaphoreType.DMA,
        pltpu.SemaphoreType.DMA,
        pltpu.SemaphoreType.DMA,
        pltpu.SemaphoreType.DMA,
    ],
)
def _sc_aggregate(x2_hbm, src2d_hbm, dst2d_hbm, agg_hbm, deg_hbm,
                  sidx0, sidx1, didx0, didx1, rows_a, rows_b, rows_c,
                  ones, z1d, aggsp, degsp,
                  ssem0, ssem1, dsem0, dsem1,
                  gsem_a, gsem_b, gsem_c, asem_a, asem_b, asem_c, degsem):
    c = lax.axis_index("c")
    s = lax.axis_index("s")

    def _zrow(r, carry):
        for j in range(HALF // 16):
            rows_a[r, pl.ds(j * 16, 16)] = jnp.zeros((16,), jnp.float32)
        return carry
    lax.fori_loop(0, EB, _zrow, 0)

    def _zflat(k, carry):
        z1d[pl.ds(k * 16, 16)] = jnp.zeros((16,), jnp.float32)
        return carry
    lax.fori_loop(0, 640 // 16, _zflat, 0)

    for j in range(EB // 16):
        ones[pl.ds(j * 16, 16)] = jnp.ones((16,), jnp.float32)

    def _zsp(i, carry):
        pltpu.sync_copy(rows_a, aggsp.at[pl.ds(s * CPT + i * EB, EB)])
        return carry
    nfull = CPT // EB
    lax.fori_loop(0, nfull, _zsp, 0)
    rem = CPT - nfull * EB
    pltpu.sync_copy(rows_a.at[pl.ds(0, rem)],
                    aggsp.at[pl.ds(s * CPT + nfull * EB, rem)])

    @pl.when(s < 15)
    def _():
        pltpu.sync_copy(z1d, degsp.at[pl.ds(s * 640, 640)])

    @pl.when(s == 15)
    def _():
        pltpu.sync_copy(z1d.at[pl.ds(0, 512)], degsp.at[pl.ds(9600, 512)])

    plsc.subcore_barrier()

    row0 = s * RPT

    def _idx_start(k, sb, db, ssem, dsem):
        pltpu.async_copy(src2d_hbm.at[pl.ds(row0 + k * CH, CH)], sb, ssem)
        pltpu.async_copy(dst2d_hbm.at[pl.ds(row0 + k * CH, CH)], db, dsem)

    def _idx_wait(k, sb, db, ssem, dsem):
        pltpu.make_async_copy(
            src2d_hbm.at[pl.ds(row0 + k * CH, CH)], sb, ssem).wait()
        pltpu.make_async_copy(
            dst2d_hbm.at[pl.ds(row0 + k * CH, CH)], db, dsem).wait()

    def _addc(sb):
        def body(r, carry):
            for j in range(EB // 16):
                sl = pl.ds(j * 16, 16)
                sb[r, sl] = sb[r, sl] + c
            return carry
        lax.fori_loop(0, CH, body, 0)

    bufs = (rows_a, rows_b, rows_c)
    gsems = (gsem_a, gsem_b, gsem_c)
    asems = (asem_a, asem_b, asem_c)

    def _g_start(sb, rr, i):
        pltpu.async_copy(x2_hbm.at[sb.at[rr]], bufs[i], gsems[i])

    def _g_wait(sb, rr, i):
        pltpu.make_async_copy(x2_hbm.at[sb.at[rr]], bufs[i], gsems[i]).wait()

    def _s_start(db, rr, i):
        pltpu.async_copy(bufs[i], aggsp.at[db.at[rr]], asems[i], add=True)

        @pl.when(c == 0)
        def _():
            pltpu.async_copy(ones, degsp.at[db.at[rr]], degsem, add=True)

    def _s_wait(db, rr, i):
        pltpu.make_async_copy(bufs[i], aggsp.at[db.at[rr]], asems[i]).wait()

    def _deg_drain(db):
        @pl.when(c == 0)
        def _():
            for rr in range(CH):
                pltpu.make_async_copy(ones, degsp.at[db.at[rr]],
                                      degsem).wait()

    def _chunk(sb, db):
        _g_start(sb, 0, 0)
        _g_start(sb, 1, 1)
        _g_start(sb, 2, 2)

        def _triple(t, carry):
            rr = 3 * t
            _g_wait(sb, rr, 0)
            _s_start(db, rr, 0)
            _g_wait(sb, rr + 1, 1)
            _s_start(db, rr + 1, 1)
            _s_wait(db, rr, 0)
            _g_start(sb, rr + 3, 0)
            _g_wait(sb, rr + 2, 2)
            _s_start(db, rr + 2, 2)
            _s_wait(db, rr + 1, 1)
            _g_start(sb, rr + 4, 1)
            _s_wait(db, rr + 2, 2)
            _g_start(sb, rr + 5, 2)
            return carry

        lax.fori_loop(0, 4, _triple, 0)
        _g_wait(sb, 12, 0)
        _s_start(db, 12, 0)
        _g_wait(sb, 13, 1)
        _s_start(db, 13, 1)
        _s_wait(db, 12, 0)
        _g_start(sb, 15, 0)
        _g_wait(sb, 14, 2)
        _s_start(db, 14, 2)
        _s_wait(db, 13, 1)
        _s_wait(db, 14, 2)
        _g_wait(sb, 15, 0)
        _s_start(db, 15, 0)
        _s_wait(db, 15, 0)
        _deg_drain(db)

    _idx_start(0, sidx0, didx0, ssem0, dsem0)
    _idx_start(1, sidx1, didx1, ssem1, dsem1)

    def _two_chunks(m, carry):
        k0 = 2 * m
        _idx_wait(k0, sidx0, didx0, ssem0, dsem0)
        _addc(sidx0)
        _chunk(sidx0, didx0)

        @pl.when(k0 + 2 < NCH)
        def _():
            _idx_start(k0 + 2, sidx0, didx0, ssem0, dsem0)

        _idx_wait(k0 + 1, sidx1, didx1, ssem1, dsem1)
        _addc(sidx1)
        _chunk(sidx1, didx1)

        @pl.when(k0 + 3 < NCH)
        def _():
            _idx_start(k0 + 3, sidx1, didx1, ssem1, dsem1)
        return carry

    lax.fori_loop(0, NCH // 2, _two_chunks, 0)

    plsc.subcore_barrier()

    pltpu.sync_copy(aggsp.at[pl.ds(s * CPT, CPT)],
                    agg_hbm.at[c, pl.ds(s * CPT, CPT)])

    @pl.when((c == 0) & (s < 15))
    def _():
        pltpu.sync_copy(degsp.at[pl.ds(s * 640, 640)],
                        deg_hbm.at[pl.ds(s * 640, 640)])

    @pl.when((c == 0) & (s == 15))
    def _():
        pltpu.sync_copy(degsp.at[pl.ds(9600, 512)],
                        deg_hbm.at[pl.ds(9600, 512)])


BN = 1000


def _tc_body(x_ref, a0_ref, a1_ref, deg_ref, wt_ref, b_ref, o_ref):
    xb = x_ref[...]
    a0 = a0_ref[0]
    a1 = a1_ref[0]
    deg = deg_ref[...]
    norm = jnp.where(deg > 0, 1.0 / deg, 0.0)
    wt = wt_ref[...]
    y = jnp.dot(xb, wt[:FEAT], preferred_element_type=jnp.float32)
    y = y + jnp.dot(a0 * norm, wt[FEAT:FEAT + HALF],
                    preferred_element_type=jnp.float32)
    y = y + jnp.dot(a1 * norm, wt[FEAT + HALF:],
                    preferred_element_type=jnp.float32)
    y = y + b_ref[...]
    mean = jnp.mean(y, axis=1, keepdims=True)
    yc = y - mean
    var = jnp.mean(yc * yc, axis=1, keepdims=True)
    o_ref[...] = yc * lax.rsqrt(var + 1e-5)


def _tc_linear_ln(x, agg2, deg2, wt, b2):
    grid = (N_NODES // BN,)
    return pl.pallas_call(
        _tc_body,
        grid=grid,
        in_specs=[
            pl.BlockSpec((BN, FEAT), lambda i: (i, 0)),
            pl.BlockSpec((1, BN, HALF), lambda i: (0, i, 0)),
            pl.BlockSpec((1, BN, HALF), lambda i: (1, i, 0)),
            pl.BlockSpec((BN, 1), lambda i: (i, 0)),
            pl.BlockSpec((2 * FEAT, OUT_F), lambda i: (0, 0)),
            pl.BlockSpec((1, OUT_F), lambda i: (0, 0)),
        ],
        out_specs=pl.BlockSpec((BN, OUT_F), lambda i: (i, 0)),
        out_shape=jax.ShapeDtypeStruct((N_NODES, OUT_F), jnp.float32),
    )(x, agg2, agg2, deg2, wt, b2)


def kernel(x, edge_index, W, b):
    src = edge_index[0].astype(jnp.int32)
    dst = edge_index[1].astype(jnp.int32)
    npad_e = EPAD - N_EDGES
    pad_src2 = (jnp.arange(npad_e, dtype=jnp.int32) % N_NODES) * 2
    pad_dst = N_NODES + (jnp.arange(npad_e, dtype=jnp.int32) % (NPAD - N_NODES))
    src2d = jnp.concatenate([src * 2, pad_src2]).reshape(EPAD // EB, EB)
    dst2d = jnp.concatenate([dst, pad_dst]).reshape(EPAD // EB, EB)
    x2 = x.reshape(2 * N_NODES, HALF)
    agg2, deg = _sc_aggregate(x2, src2d, dst2d)
    wt = W.T
    deg2 = deg.reshape(NPAD, 1)
    b2 = b.reshape(1, OUT_F)
    return _tc_linear_ln(x, agg2, deg2, wt, b2)


# speedup vs baseline: 1.0205x; 1.0205x over previous
import functools

import jax
import jax.numpy as jnp
from jax import lax
from jax.experimental import pallas as pl
from jax.experimental.pallas import tpu as pltpu
from jax.experimental.pallas import tpu_sc as plsc

N_NODES = 10000
N_EDGES = 160000
FEAT = 256
HALF = 128
OUT_F = 256

NPAD = 10112
CPT = NPAD // 16
EB = 64
RPT = 160
CH = 16
NCH = RPT // CH
EPAD = 16 * RPT * EB


_sc_mesh = plsc.VectorSubcoreMesh(core_axis_name="c", subcore_axis_name="s")


@functools.partial(
    pl.kernel,
    out_type=[
        jax.ShapeDtypeStruct((2, NPAD, HALF), jnp.float32),
        jax.ShapeDtypeStruct((2, NPAD), jnp.float32),
    ],
    mesh=_sc_mesh,
    scratch_types=[
        pltpu.VMEM((CH, EB), jnp.int32),
        pltpu.VMEM((CH, EB), jnp.int32),
        pltpu.VMEM((CH, EB), jnp.int32),
        pltpu.VMEM((CH, EB), jnp.int32),
        pltpu.VMEM((EB, HALF), jnp.float32),
        pltpu.VMEM((EB, HALF), jnp.float32),
        pltpu.VMEM((EB, HALF), jnp.float32),
        pltpu.VMEM((EB,), jnp.float32),
        pltpu.VMEM((640,), jnp.float32),
        pltpu.VMEM_SHARED((NPAD, HALF), jnp.float32),
        pltpu.VMEM_SHARED((NPAD,), jnp.float32),
        pltpu.SemaphoreType.DMA,
        pltpu.SemaphoreType.DMA,
        pltpu.SemaphoreType.DMA,
        pltpu.SemaphoreType.DMA,
        pltpu.SemaphoreType.DMA,
        pltpu.SemaphoreType.DMA,
        pltpu.SemaphoreType.DMA,
        pltpu.SemaphoreType.DMA,
        pltpu.SemaphoreType.DMA,
        pltpu.SemaphoreType.DMA,
        pltpu.SemaphoreType.DMA,
    ],
)
def _sc_aggregate(x2_hbm, src2d_hbm, dst2d_hbm, agg_hbm, deg_hbm,
                  sidx0, sidx1, didx0, didx1, rows_a, rows_b, rows_c,
                  ones, z1d, aggsp, degsp,
                  ssem0, ssem1, dsem0, dsem1,
                  gsem_a, gsem_b, gsem_c, asem_a, asem_b, asem_c, degsem):
    c = lax.axis_index("c")
    s = lax.axis_index("s")

    def _zrow(r, carry):
        for j in range(HALF // 16):
            rows_a[r, pl.ds(j * 16, 16)] = jnp.zeros((16,), jnp.float32)
        return carry
    lax.fori_loop(0, EB, _zrow, 0)

    def _zflat(k, carry):
        z1d[pl.ds(k * 16, 16)] = jnp.zeros((16,), jnp.float32)
        return carry
    lax.fori_loop(0, 640 // 16, _zflat, 0)

    for j in range(EB // 16):
        ones[pl.ds(j * 16, 16)] = jnp.ones((16,), jnp.float32)

    def _zsp(i, carry):
        pltpu.sync_copy(rows_a, aggsp.at[pl.ds(s * CPT + i * EB, EB)])
        return carry
    nfull = CPT // EB
    lax.fori_loop(0, nfull, _zsp, 0)
    rem = CPT - nfull * EB
    pltpu.sync_copy(rows_a.at[pl.ds(0, rem)],
                    aggsp.at[pl.ds(s * CPT + nfull * EB, rem)])

    @pl.when(s < 15)
    def _():
        pltpu.sync_copy(z1d, degsp.at[pl.ds(s * 640, 640)])

    @pl.when(s == 15)
    def _():
        pltpu.sync_copy(z1d.at[pl.ds(0, 512)], degsp.at[pl.ds(9600, 512)])

    plsc.subcore_barrier()

    row0 = s * RPT

    def _idx_start(k, sb, db, ssem, dsem):
        pltpu.async_copy(src2d_hbm.at[pl.ds(row0 + k * CH, CH)], sb, ssem)
        pltpu.async_copy(dst2d_hbm.at[pl.ds(row0 + k * CH, CH)], db, dsem)

    def _idx_wait(k, sb, db, ssem, dsem):
        pltpu.make_async_copy(
            src2d_hbm.at[pl.ds(row0 + k * CH, CH)], sb, ssem).wait()
        pltpu.make_async_copy(
            dst2d_hbm.at[pl.ds(row0 + k * CH, CH)], db, dsem).wait()

    def _addc(sb):
        def body(r, carry):
            for j in range(EB // 16):
                sl = pl.ds(j * 16, 16)
                sb[r, sl] = sb[r, sl] + c
            return carry
        lax.fori_loop(0, CH, body, 0)

    bufs = (rows_a, rows_b, rows_c)
    gsems = (gsem_a, gsem_b, gsem_c)
    asems = (asem_a, asem_b, asem_c)

    def _g_start(sb, rr, i):
        pltpu.async_copy(x2_hbm.at[sb.at[rr]], bufs[i], gsems[i])

    def _g_wait(sb, rr, i):
        pltpu.make_async_copy(x2_hbm.at[sb.at[rr]], bufs[i], gsems[i]).wait()

    def _s_start(db, rr, i, do_deg):
        pltpu.async_copy(bufs[i], aggsp.at[db.at[rr]], asems[i], add=True)

        @pl.when(do_deg)
        def _():
            pltpu.async_copy(ones, degsp.at[db.at[rr]], degsem, add=True)

    def _s_wait(db, rr, i):
        pltpu.make_async_copy(bufs[i], aggsp.at[db.at[rr]], asems[i]).wait()

    def _deg_drain(db, do_deg):
        @pl.when(do_deg)
        def _():
            for rr in range(CH):
                pltpu.make_async_copy(ones, degsp.at[db.at[rr]],
                                      degsem).wait()

    def _chunk(sb, db, do_deg):
        _g_start(sb, 0, 0)
        _g_start(sb, 1, 1)
        _g_start(sb, 2, 2)

        def _triple(t, carry):
            rr = 3 * t
            _g_wait(sb, rr, 0)
            _s_start(db, rr, 0, do_deg)
            _g_wait(sb, rr + 1, 1)
            _s_start(db, rr + 1, 1, do_deg)
            _s_wait(db, rr, 0)
            _g_start(sb, rr + 3, 0)
            _g_wait(sb, rr + 2, 2)
            _s_start(db, rr + 2, 2, do_deg)
            _s_wait(db, rr + 1, 1)
            _g_start(sb, rr + 4, 1)
            _s_wait(db, rr + 2, 2)
            _g_start(sb, rr + 5, 2)
            return carry

        lax.fori_loop(0, 4, _triple, 0)
        _g_wait(sb, 12, 0)
        _s_start(db, 12, 0, do_deg)
        _g_wait(sb, 13, 1)
        _s_start(db, 13, 1, do_deg)
        _s_wait(db, 12, 0)
        _g_start(sb, 15, 0)
        _g_wait(sb, 14, 2)
        _s_start(db, 14, 2, do_deg)
        _s_wait(db, 13, 1)
        _s_wait(db, 14, 2)
        _g_wait(sb, 15, 0)
        _s_start(db, 15, 0, do_deg)
        _s_wait(db, 15, 0)
        _deg_drain(db, do_deg)

    _idx_start(0, sidx0, didx0, ssem0, dsem0)
    _idx_start(1, sidx1, didx1, ssem1, dsem1)

    def _two_chunks(m, carry):
        k0 = 2 * m
        _idx_wait(k0, sidx0, didx0, ssem0, dsem0)
        _addc(sidx0)
        _chunk(sidx0, didx0, (c == 0) == (k0 < NCH // 2))

        @pl.when(k0 + 2 < NCH)
        def _():
            _idx_start(k0 + 2, sidx0, didx0, ssem0, dsem0)

        _idx_wait(k0 + 1, sidx1, didx1, ssem1, dsem1)
        _addc(sidx1)
        _chunk(sidx1, didx1, (c == 0) == (k0 + 1 < NCH // 2))

        @pl.when(k0 + 3 < NCH)
        def _():
            _idx_start(k0 + 3, sidx1, didx1, ssem1, dsem1)
        return carry

    lax.fori_loop(0, NCH // 2, _two_chunks, 0)

    plsc.subcore_barrier()

    pltpu.sync_copy(aggsp.at[pl.ds(s * CPT, CPT)],
                    agg_hbm.at[c, pl.ds(s * CPT, CPT)])

    @pl.when(s < 15)
    def _():
        pltpu.sync_copy(degsp.at[pl.ds(s * 640, 640)],
                        deg_hbm.at[c, pl.ds(s * 640, 640)])

    @pl.when(s == 15)
    def _():
        pltpu.sync_copy(degsp.at[pl.ds(9600, 512)],
                        deg_hbm.at[c, pl.ds(9600, 512)])


BN = 1000


def _tc1_body(x_ref, w1_ref, b_ref, o_ref):
    o_ref[...] = jnp.dot(x_ref[...], w1_ref[...],
                         preferred_element_type=jnp.float32) + b_ref[...]


def _tc_xw1(x, w1t, b2):
    return pl.pallas_call(
        _tc1_body,
        grid=(N_NODES // BN,),
        in_specs=[
            pl.BlockSpec((BN, FEAT), lambda i: (i, 0)),
            pl.BlockSpec((FEAT, OUT_F), lambda i: (0, 0)),
            pl.BlockSpec((1, OUT_F), lambda i: (0, 0)),
        ],
        out_specs=pl.BlockSpec((BN, OUT_F), lambda i: (i, 0)),
        out_shape=jax.ShapeDtypeStruct((N_NODES, OUT_F), jnp.float32),
    )(x, w1t, b2)


def _tc2_body(y1_ref, a0_ref, a1_ref, deg_ref, w2_ref, o_ref):
    a0 = a0_ref[0]
    a1 = a1_ref[0]
    deg = deg_ref[...]
    norm = jnp.where(deg > 0, 1.0 / deg, 0.0)
    w2 = w2_ref[...]
    y = y1_ref[...]
    y = y + jnp.dot(a0 * norm, w2[:HALF], preferred_element_type=jnp.float32)
    y = y + jnp.dot(a1 * norm, w2[HALF:], preferred_element_type=jnp.float32)
    mean = jnp.mean(y, axis=1, keepdims=True)
    yc = y - mean
    var = jnp.mean(yc * yc, axis=1, keepdims=True)
    o_ref[...] = yc * lax.rsqrt(var + 1e-5)


def _tc_agg_ln(y1, agg2, deg2, w2t):
    return pl.pallas_call(
        _tc2_body,
        grid=(N_NODES // BN,),
        in_specs=[
            pl.BlockSpec((BN, OUT_F), lambda i: (i, 0)),
            pl.BlockSpec((1, BN, HALF), lambda i: (0, i, 0)),
            pl.BlockSpec((1, BN, HALF), lambda i: (1, i, 0)),
            pl.BlockSpec((BN, 1), lambda i: (i, 0)),
            pl.BlockSpec((2 * HALF, OUT_F), lambda i: (0, 0)),
        ],
        out_specs=pl.BlockSpec((BN, OUT_F), lambda i: (i, 0)),
        out_shape=jax.ShapeDtypeStruct((N_NODES, OUT_F), jnp.float32),
    )(y1, agg2, agg2, deg2, w2t)


def kernel(x, edge_index, W, b):
    src = edge_index[0].astype(jnp.int32)
    dst = edge_index[1].astype(jnp.int32)
    npad_e = EPAD - N_EDGES
    pad_src2 = (jnp.arange(npad_e, dtype=jnp.int32) % N_NODES) * 2
    pad_dst = N_NODES + (jnp.arange(npad_e, dtype=jnp.int32) % (NPAD - N_NODES))
    src2d = jnp.concatenate([src * 2, pad_src2]).reshape(EPAD // EB, EB)
    dst2d = jnp.concatenate([dst, pad_dst]).reshape(EPAD // EB, EB)
    x2 = x.reshape(2 * N_NODES, HALF)
    agg2, degp = _sc_aggregate(x2, src2d, dst2d)
    wt = W.T
    b2 = b.reshape(1, OUT_F)
    y1 = _tc_xw1(x, wt[:FEAT], b2)
    deg2 = (degp[0] + degp[1]).reshape(NPAD, 1)
    return _tc_agg_ln(y1, agg2, deg2, wt[FEAT:])


# speedup vs baseline: 1.0802x; 1.0585x over previous
import functools

import jax
import jax.numpy as jnp
from jax import lax
from jax.experimental import pallas as pl
from jax.experimental.pallas import tpu as pltpu
from jax.experimental.pallas import tpu_sc as plsc

N_NODES = 10000
N_EDGES = 160000
FEAT = 256
HALF = 128
OUT_F = 256

NPAD = 10112
CPT = NPAD // 16
EB = 64
RPT = 160
CH = 16
NCH = RPT // CH
EPAD = 16 * RPT * EB


_sc_mesh = plsc.VectorSubcoreMesh(core_axis_name="c", subcore_axis_name="s")


@functools.partial(
    pl.kernel,
    out_type=[
        jax.ShapeDtypeStruct((2, NPAD, HALF), jnp.float32),
        jax.ShapeDtypeStruct((2, NPAD), jnp.float32),
    ],
    mesh=_sc_mesh,
    scratch_types=[
        pltpu.VMEM((2 * CH, EB), jnp.int32),
        pltpu.VMEM((2 * CH, EB), jnp.int32),
        pltpu.VMEM((EB, HALF), jnp.float32),
        pltpu.VMEM((EB, HALF), jnp.float32),
        pltpu.VMEM((EB, HALF), jnp.float32),
        pltpu.VMEM((EB,), jnp.float32),
        pltpu.VMEM((640,), jnp.float32),
        pltpu.VMEM_SHARED((NPAD, HALF), jnp.float32),
        pltpu.VMEM_SHARED((NPAD,), jnp.float32),
        pltpu.SemaphoreType.DMA,
        pltpu.SemaphoreType.DMA,
        pltpu.SemaphoreType.DMA,
        pltpu.SemaphoreType.DMA,
        pltpu.SemaphoreType.DMA,
        pltpu.SemaphoreType.DMA,
        pltpu.SemaphoreType.DMA,
        pltpu.SemaphoreType.DMA,
        pltpu.SemaphoreType.DMA,
        pltpu.SemaphoreType.DMA,
        pltpu.SemaphoreType.DMA,
    ],
)
def _sc_aggregate(x2_hbm, src2d_hbm, dst2d_hbm, agg_hbm, deg_hbm,
                  sring, dring, rows_a, rows_b, rows_c,
                  ones, z1d, aggsp, degsp,
                  ssem0, ssem1, dsem0, dsem1,
                  gsem_a, gsem_b, gsem_c, asem_a, asem_b, asem_c, degsem):
    c = lax.axis_index("c")
    s = lax.axis_index("s")

    def _zrow(r, carry):
        for j in range(HALF // 16):
            rows_a[r, pl.ds(j * 16, 16)] = jnp.zeros((16,), jnp.float32)
        return carry
    lax.fori_loop(0, EB, _zrow, 0)

    def _zflat(k, carry):
        z1d[pl.ds(k * 16, 16)] = jnp.zeros((16,), jnp.float32)
        return carry
    lax.fori_loop(0, 640 // 16, _zflat, 0)

    for j in range(EB // 16):
        ones[pl.ds(j * 16, 16)] = jnp.ones((16,), jnp.float32)

    def _zsp(i, carry):
        pltpu.sync_copy(rows_a, aggsp.at[pl.ds(s * CPT + i * EB, EB)])
        return carry
    nfull = CPT // EB
    lax.fori_loop(0, nfull, _zsp, 0)
    rem = CPT - nfull * EB
    pltpu.sync_copy(rows_a.at[pl.ds(0, rem)],
                    aggsp.at[pl.ds(s * CPT + nfull * EB, rem)])

    @pl.when(s < 15)
    def _():
        pltpu.sync_copy(z1d, degsp.at[pl.ds(s * 640, 640)])

    @pl.when(s == 15)
    def _():
        pltpu.sync_copy(z1d.at[pl.ds(0, 512)], degsp.at[pl.ds(9600, 512)])

    plsc.subcore_barrier()

    row0 = s * RPT

    def _idx_start(k):
        h = (k % 2) * CH
        ssem = ssem0 if k % 2 == 0 else ssem1
        dsem = dsem0 if k % 2 == 0 else dsem1
        pltpu.async_copy(src2d_hbm.at[pl.ds(row0 + k * CH, CH)],
                         sring.at[pl.ds(h, CH)], ssem)
        pltpu.async_copy(dst2d_hbm.at[pl.ds(row0 + k * CH, CH)],
                         dring.at[pl.ds(h, CH)], dsem)

    def _idx_wait(k):
        h = (k % 2) * CH
        ssem = ssem0 if k % 2 == 0 else ssem1
        dsem = dsem0 if k % 2 == 0 else dsem1
        pltpu.make_async_copy(src2d_hbm.at[pl.ds(row0 + k * CH, CH)],
                              sring.at[pl.ds(h, CH)], ssem).wait()
        pltpu.make_async_copy(dst2d_hbm.at[pl.ds(row0 + k * CH, CH)],
                              dring.at[pl.ds(h, CH)], dsem).wait()

    def _addc(k):
        h = (k % 2) * CH

        def body(r, carry):
            for j in range(EB // 16):
                sl = pl.ds(j * 16, 16)
                sring[h + r, sl] = sring[h + r, sl] + c
            return carry
        lax.fori_loop(0, CH, body, 0)

    bufs = (rows_a, rows_b, rows_c)
    gsems = (gsem_a, gsem_b, gsem_c)
    asems = (asem_a, asem_b, asem_c)

    def _ring(bb):
        return lax.rem(bb, 2 * CH)

    def _g_start(bb, i):
        pltpu.async_copy(x2_hbm.at[sring.at[_ring(bb)]], bufs[i], gsems[i])

    def _g_wait(bb, i):
        pltpu.make_async_copy(x2_hbm.at[sring.at[_ring(bb)]],
                              bufs[i], gsems[i]).wait()

    def _s_start(bb, i):
        r = _ring(bb)
        pltpu.async_copy(bufs[i], aggsp.at[dring.at[r]], asems[i], add=True)
        do_deg = (c == 0) == (bb // CH < NCH // 2)

        @pl.when(do_deg)
        def _():
            pltpu.async_copy(ones, degsp.at[dring.at[r]], degsem, add=True)

    def _s_wait(bb, i):
        pltpu.make_async_copy(bufs[i], aggsp.at[dring.at[_ring(bb)]],
                              asems[i]).wait()

    def _deg_drain(n):
        for _ in range(n):
            pltpu.make_async_copy(ones, degsp.at[dring.at[0]], degsem).wait()

    _WAITS = {4: 1, 9: 2, 15: 3, 20: 4, 25: 5, 31: 6, 36: 7, 41: 8, 47: 9}
    _ISSUES = {6: 2, 11: 3, 16: 4, 22: 5, 27: 6, 32: 7, 38: 8, 43: 9}

    _idx_start(0)
    _idx_start(1)
    _idx_wait(0)
    _addc(0)
    _g_start(0, 0)
    _g_start(1, 1)
    _g_start(2, 2)

    def _triple(t, carry):
        for tt in sorted(set(_WAITS) | set(_ISSUES)):
            @pl.when(t == tt)
            def _(tt=tt):
                if tt in _ISSUES:
                    j = _ISSUES[tt]
                    @pl.when((c == 0) == (j - 2 < NCH // 2))
                    def _():
                        _deg_drain(CH)
                    _idx_start(j)
                if tt in _WAITS:
                    j = _WAITS[tt]
                    _idx_wait(j)
                    _addc(j)

        rr = 3 * t
        _g_wait(rr, 0)
        _s_start(rr, 0)
        _g_wait(rr + 1, 1)
        _s_start(rr + 1, 1)
        _s_wait(rr, 0)
        _g_start(rr + 3, 0)
        _g_wait(rr + 2, 2)
        _s_start(rr + 2, 2)
        _s_wait(rr + 1, 1)
        _g_start(rr + 4, 1)
        _s_wait(rr + 2, 2)
        _g_start(rr + 5, 2)
        return carry

    lax.fori_loop(0, (RPT - 4) // 3, _triple, 0)

    _g_wait(RPT - 4, 0)
    _s_start(RPT - 4, 0)
    _g_wait(RPT - 3, 1)
    _s_start(RPT - 3, 1)
    _s_wait(RPT - 4, 0)
    _g_start(RPT - 1, 0)
    _g_wait(RPT - 2, 2)
    _s_start(RPT - 2, 2)
    _s_wait(RPT - 3, 1)
    _s_wait(RPT - 2, 2)
    _g_wait(RPT - 1, 0)
    _s_start(RPT - 1, 0)
    _s_wait(RPT - 1, 0)

    @pl.when(c == 1)
    def _():
        _deg_drain(2 * CH)

    plsc.subcore_barrier()

    pltpu.sync_copy(aggsp.at[pl.ds(s * CPT, CPT)],
                    agg_hbm.at[c, pl.ds(s * CPT, CPT)])

    @pl.when(s < 15)
    def _():
        pltpu.sync_copy(degsp.at[pl.ds(s * 640, 640)],
                        deg_hbm.at[c, pl.ds(s * 640, 640)])

    @pl.when(s == 15)
    def _():
        pltpu.sync_copy(degsp.at[pl.ds(9600, 512)],
                        deg_hbm.at[c, pl.ds(9600, 512)])


BN = 1000


def _tc1_body(x_ref, w1_ref, b_ref, o_ref):
    o_ref[...] = jnp.dot(x_ref[...], w1_ref[...],
                         preferred_element_type=jnp.float32) + b_ref[...]


def _tc_xw1(x, w1t, b2):
    return pl.pallas_call(
        _tc1_body,
        grid=(N_NODES // BN,),
        in_specs=[
            pl.BlockSpec((BN, FEAT), lambda i: (i, 0)),
            pl.BlockSpec((FEAT, OUT_F), lambda i: (0, 0)),
            pl.BlockSpec((1, OUT_F), lambda i: (0, 0)),
        ],
        out_specs=pl.BlockSpec((BN, OUT_F), lambda i: (i, 0)),
        out_shape=jax.ShapeDtypeStruct((N_NODES, OUT_F), jnp.float32),
    )(x, w1t, b2)


def _tc2_body(y1_ref, a0_ref, a1_ref, deg_ref, w2_ref, o_ref):
    a0 = a0_ref[0]
    a1 = a1_ref[0]
    deg = deg_ref[...]
    norm = jnp.where(deg > 0, 1.0 / deg, 0.0)
    w2 = w2_ref[...]
    y = y1_ref[...]
    y = y + jnp.dot(a0 * norm, w2[:HALF], preferred_element_type=jnp.float32)
    y = y + jnp.dot(a1 * norm, w2[HALF:], preferred_element_type=jnp.float32)
    mean = jnp.mean(y, axis=1, keepdims=True)
    yc = y - mean
    var = jnp.mean(yc * yc, axis=1, keepdims=True)
    o_ref[...] = yc * lax.rsqrt(var + 1e-5)


def _tc_agg_ln(y1, agg2, deg2, w2t):
    return pl.pallas_call(
        _tc2_body,
        grid=(N_NODES // BN,),
        in_specs=[
            pl.BlockSpec((BN, OUT_F), lambda i: (i, 0)),
            pl.BlockSpec((1, BN, HALF), lambda i: (0, i, 0)),
            pl.BlockSpec((1, BN, HALF), lambda i: (1, i, 0)),
            pl.BlockSpec((BN, 1), lambda i: (i, 0)),
            pl.BlockSpec((2 * HALF, OUT_F), lambda i: (0, 0)),
        ],
        out_specs=pl.BlockSpec((BN, OUT_F), lambda i: (i, 0)),
        out_shape=jax.ShapeDtypeStruct((N_NODES, OUT_F), jnp.float32),
    )(y1, agg2, agg2, deg2, w2t)


def kernel(x, edge_index, W, b):
    src = edge_index[0].astype(jnp.int32)
    dst = edge_index[1].astype(jnp.int32)
    npad_e = EPAD - N_EDGES
    pad_src2 = (jnp.arange(npad_e, dtype=jnp.int32) % N_NODES) * 2
    pad_dst = N_NODES + (jnp.arange(npad_e, dtype=jnp.int32) % (NPAD - N_NODES))
    src2d = jnp.concatenate([src * 2, pad_src2]).reshape(EPAD // EB, EB)
    dst2d = jnp.concatenate([dst, pad_dst]).reshape(EPAD // EB, EB)
    x2 = x.reshape(2 * N_NODES, HALF)
    agg2, degp = _sc_aggregate(x2, src2d, dst2d)
    wt = W.T
    b2 = b.reshape(1, OUT_F)
    y1 = _tc_xw1(x, wt[:FEAT], b2)
    deg2 = (degp[0] + degp[1]).reshape(NPAD, 1)
    return _tc_agg_ln(y1, agg2, deg2, wt[FEAT:])


# speedup vs baseline: 1.0825x; 1.0022x over previous
import functools

import jax
import jax.numpy as jnp
from jax import lax
from jax.experimental import pallas as pl
from jax.experimental.pallas import tpu as pltpu
from jax.experimental.pallas import tpu_sc as plsc

N_NODES = 10000
N_EDGES = 160000
FEAT = 256
HALF = 128
OUT_F = 256

NPAD = 10112
CPT = NPAD // 16
EB = 64
RPT = 160
CH = 16
NCH = RPT // CH
EPAD = 16 * RPT * EB


_sc_mesh = plsc.VectorSubcoreMesh(core_axis_name="c", subcore_axis_name="s")


@functools.partial(
    pl.kernel,
    out_type=[
        jax.ShapeDtypeStruct((2, NPAD, HALF), jnp.float32),
        jax.ShapeDtypeStruct((2, NPAD), jnp.float32),
    ],
    mesh=_sc_mesh,
    scratch_types=[
        pltpu.VMEM((2 * CH, EB), jnp.int32),
        pltpu.VMEM((2 * CH, EB), jnp.int32),
        pltpu.VMEM((EB, HALF), jnp.float32),
        pltpu.VMEM((EB, HALF), jnp.float32),
        pltpu.VMEM((EB, HALF), jnp.float32),
        pltpu.VMEM((EB,), jnp.float32),
        pltpu.VMEM((640,), jnp.float32),
        pltpu.VMEM_SHARED((NPAD, HALF), jnp.float32),
        pltpu.VMEM_SHARED((NPAD,), jnp.float32),
        pltpu.SemaphoreType.DMA,
        pltpu.SemaphoreType.DMA,
        pltpu.SemaphoreType.DMA,
        pltpu.SemaphoreType.DMA,
        pltpu.SemaphoreType.DMA,
        pltpu.SemaphoreType.DMA,
        pltpu.SemaphoreType.DMA,
        pltpu.SemaphoreType.DMA,
        pltpu.SemaphoreType.DMA,
        pltpu.SemaphoreType.DMA,
        pltpu.SemaphoreType.DMA,
    ],
)
def _sc_aggregate(x2_hbm, src2d_hbm, dst2d_hbm, agg_hbm, deg_hbm,
                  sring, dring, rows_a, rows_b, rows_c,
                  ones, z1d, aggsp, degsp,
                  ssem0, ssem1, dsem0, dsem1,
                  gsem_a, gsem_b, gsem_c, asem_a, asem_b, asem_c, degsem):
    c = lax.axis_index("c")
    s = lax.axis_index("s")

    def _zrow(r, carry):
        for j in range(HALF // 16):
            rows_a[r, pl.ds(j * 16, 16)] = jnp.zeros((16,), jnp.float32)
        return carry
    lax.fori_loop(0, EB, _zrow, 0)

    def _zflat(k, carry):
        z1d[pl.ds(k * 16, 16)] = jnp.zeros((16,), jnp.float32)
        return carry
    lax.fori_loop(0, 640 // 16, _zflat, 0)

    for j in range(EB // 16):
        ones[pl.ds(j * 16, 16)] = jnp.ones((16,), jnp.float32)

    row0 = s * RPT

    def _idx_start(k):
        h = (k % 2) * CH
        ssem = ssem0 if k % 2 == 0 else ssem1
        dsem = dsem0 if k % 2 == 0 else dsem1
        pltpu.async_copy(src2d_hbm.at[pl.ds(row0 + k * CH, CH)],
                         sring.at[pl.ds(h, CH)], ssem)
        pltpu.async_copy(dst2d_hbm.at[pl.ds(row0 + k * CH, CH)],
                         dring.at[pl.ds(h, CH)], dsem)

    def _idx_wait(k):
        h = (k % 2) * CH
        ssem = ssem0 if k % 2 == 0 else ssem1
        dsem = dsem0 if k % 2 == 0 else dsem1
        pltpu.make_async_copy(src2d_hbm.at[pl.ds(row0 + k * CH, CH)],
                              sring.at[pl.ds(h, CH)], ssem).wait()
        pltpu.make_async_copy(dst2d_hbm.at[pl.ds(row0 + k * CH, CH)],
                              dring.at[pl.ds(h, CH)], dsem).wait()

    def _addc(k):
        h = (k % 2) * CH

        def body(r, carry):
            for j in range(EB // 16):
                sl = pl.ds(j * 16, 16)
                sring[h + r, sl] = sring[h + r, sl] + c
            return carry
        lax.fori_loop(0, CH, body, 0)

    bufs = (rows_a, rows_b, rows_c)
    gsems = (gsem_a, gsem_b, gsem_c)
    asems = (asem_a, asem_b, asem_c)

    def _ring(bb):
        return lax.rem(bb, 2 * CH)

    def _g_start(bb, i):
        pltpu.async_copy(x2_hbm.at[sring.at[_ring(bb)]], bufs[i], gsems[i])

    def _g_wait(bb, i):
        pltpu.make_async_copy(x2_hbm.at[sring.at[_ring(bb)]],
                              bufs[i], gsems[i]).wait()

    def _s_start(bb, i):
        r = _ring(bb)
        pltpu.async_copy(bufs[i], aggsp.at[dring.at[r]], asems[i], add=True)
        do_deg = (c == 0) == (bb // CH < NCH // 2)

        @pl.when(do_deg)
        def _():
            pltpu.async_copy(ones, degsp.at[dring.at[r]], degsem, add=True)

    def _s_wait(bb, i):
        pltpu.make_async_copy(bufs[i], aggsp.at[dring.at[_ring(bb)]],
                              asems[i]).wait()

    def _deg_drain(n):
        for _ in range(n):
            pltpu.make_async_copy(ones, degsp.at[dring.at[0]], degsem).wait()

    _WAITS = {4: 1, 9: 2, 15: 3, 20: 4, 25: 5, 31: 6, 36: 7, 41: 8, 47: 9}
    _ISSUES = {6: 2, 11: 3, 16: 4, 22: 5, 27: 6, 32: 7, 38: 8, 43: 9}

    _idx_start(0)
    _idx_start(1)

    def _zsp(i, carry):
        pltpu.sync_copy(rows_a, aggsp.at[pl.ds(s * CPT + i * EB, EB)])
        return carry
    nfull = CPT // EB
    lax.fori_loop(0, nfull, _zsp, 0)
    rem = CPT - nfull * EB
    pltpu.sync_copy(rows_a.at[pl.ds(0, rem)],
                    aggsp.at[pl.ds(s * CPT + nfull * EB, rem)])

    @pl.when(s < 15)
    def _():
        pltpu.sync_copy(z1d, degsp.at[pl.ds(s * 640, 640)])

    @pl.when(s == 15)
    def _():
        pltpu.sync_copy(z1d.at[pl.ds(0, 512)], degsp.at[pl.ds(9600, 512)])

    _idx_wait(0)
    _addc(0)
    _g_start(0, 0)
    _g_start(1, 1)
    _g_start(2, 2)
    plsc.subcore_barrier()

    def _triple(t, carry):
        for tt in sorted(set(_WAITS) | set(_ISSUES)):
            @pl.when(t == tt)
            def _(tt=tt):
                if tt in _ISSUES:
                    j = _ISSUES[tt]
                    @pl.when((c == 0) == (j - 2 < NCH // 2))
                    def _():
                        _deg_drain(CH)
                    _idx_start(j)
                if tt in _WAITS:
                    j = _WAITS[tt]
                    _idx_wait(j)
                    _addc(j)

        rr = 3 * t
        _g_wait(rr, 0)
        _s_start(rr, 0)
        _g_wait(rr + 1, 1)
        _s_start(rr + 1, 1)
        _s_wait(rr, 0)
        _g_start(rr + 3, 0)
        _g_wait(rr + 2, 2)
        _s_start(rr + 2, 2)
        _s_wait(rr + 1, 1)
        _g_start(rr + 4, 1)
        _s_wait(rr + 2, 2)
        _g_start(rr + 5, 2)
        return carry

    lax.fori_loop(0, (RPT - 4) // 3, _triple, 0)

    _g_wait(RPT - 4, 0)
    _s_start(RPT - 4, 0)
    _g_wait(RPT - 3, 1)
    _s_start(RPT - 3, 1)
    _s_wait(RPT - 4, 0)
    _g_start(RPT - 1, 0)
    _g_wait(RPT - 2, 2)
    _s_start(RPT - 2, 2)
    _s_wait(RPT - 3, 1)
    _s_wait(RPT - 2, 2)
    _g_wait(RPT - 1, 0)
    _s_start(RPT - 1, 0)
    _s_wait(RPT - 1, 0)

    @pl.when(c == 1)
    def _():
        _deg_drain(2 * CH)

    plsc.subcore_barrier()

    pltpu.sync_copy(aggsp.at[pl.ds(s * CPT, CPT)],
                    agg_hbm.at[c, pl.ds(s * CPT, CPT)])

    @pl.when(s < 15)
    def _():
        pltpu.sync_copy(degsp.at[pl.ds(s * 640, 640)],
                        deg_hbm.at[c, pl.ds(s * 640, 640)])

    @pl.when(s == 15)
    def _():
        pltpu.sync_copy(degsp.at[pl.ds(9600, 512)],
                        deg_hbm.at[c, pl.ds(9600, 512)])


BN = 1000


def _tc_body(x_ref, a0_ref, a1_ref, deg_ref, wt_ref, b_ref, o_ref):
    xb = x_ref[...]
    a0 = a0_ref[0]
    a1 = a1_ref[0]
    deg = deg_ref[...]
    norm = jnp.where(deg > 0, 1.0 / deg, 0.0)
    wt = wt_ref[...]
    y = jnp.dot(xb, wt[:FEAT], preferred_element_type=jnp.float32)
    y = y + jnp.dot(a0 * norm, wt[FEAT:FEAT + HALF],
                    preferred_element_type=jnp.float32)
    y = y + jnp.dot(a1 * norm, wt[FEAT + HALF:],
                    preferred_element_type=jnp.float32)
    y = y + b_ref[...]
    mean = jnp.mean(y, axis=1, keepdims=True)
    yc = y - mean
    var = jnp.mean(yc * yc, axis=1, keepdims=True)
    o_ref[...] = yc * lax.rsqrt(var + 1e-5)


def _tc_linear_ln(x, agg2, deg2, wt, b2):
    grid = (N_NODES // BN,)
    return pl.pallas_call(
        _tc_body,
        grid=grid,
        in_specs=[
            pl.BlockSpec((BN, FEAT), lambda i: (i, 0)),
            pl.BlockSpec((1, BN, HALF), lambda i: (0, i, 0)),
            pl.BlockSpec((1, BN, HALF), lambda i: (1, i, 0)),
            pl.BlockSpec((BN, 1), lambda i: (i, 0)),
            pl.BlockSpec((2 * FEAT, OUT_F), lambda i: (0, 0)),
            pl.BlockSpec((1, OUT_F), lambda i: (0, 0)),
        ],
        out_specs=pl.BlockSpec((BN, OUT_F), lambda i: (i, 0)),
        out_shape=jax.ShapeDtypeStruct((N_NODES, OUT_F), jnp.float32),
    )(x, agg2, agg2, deg2, wt, b2)


def kernel(x, edge_index, W, b):
    src = edge_index[0].astype(jnp.int32)
    dst = edge_index[1].astype(jnp.int32)
    npad_e = EPAD - N_EDGES
    pad_src2 = (jnp.arange(npad_e, dtype=jnp.int32) % N_NODES) * 2
    pad_dst = N_NODES + (jnp.arange(npad_e, dtype=jnp.int32) % (NPAD - N_NODES))
    src2d = jnp.concatenate([src * 2, pad_src2]).reshape(EPAD // EB, EB)
    dst2d = jnp.concatenate([dst, pad_dst]).reshape(EPAD // EB, EB)
    x2 = x.reshape(2 * N_NODES, HALF)
    agg2, degp = _sc_aggregate(x2, src2d, dst2d)
    wt = W.T
    b2 = b.reshape(1, OUT_F)
    deg2 = (degp[0] + degp[1]).reshape(NPAD, 1)
    return _tc_linear_ln(x, agg2, deg2, wt, b2)
